# baseline (speedup 1.0000x reference)
import math

import jax
import jax.numpy as jnp
from jax import lax
from jax.experimental import pallas as pl
from jax.experimental.pallas import tpu as pltpu
from jax.experimental.pallas import tpu_sc as plsc

VOCAB = 1000
SEMEME_SIZE = 2048
D_MODEL = 128

_NC = 2
_NS = 16
_NW = _NC * _NS


_M_STEPS = 5
_M_BLOCK = VOCAB // _M_STEPS


def _table_body(w2s_ref, lut_ref, xt_ref, t_ref, idx_ref):
    i = pl.program_id(0)
    w = w2s_ref[...]
    s = jnp.sum(w, axis=1, keepdims=True) + 1e-6
    acc = jnp.dot(w, lut_ref[...], preferred_element_type=jnp.float32)
    t_ref[...] = acc * (math.sqrt(D_MODEL) / s)

    @pl.when(i == 0)
    def _():
        idx_ref[...] = xt_ref[...].reshape(-1)


def _build_table(word2sememe, lut_weight, xt):
    n_idx = xt.shape[0] * xt.shape[1]
    return pl.pallas_call(
        _table_body,
        grid=(_M_STEPS,),
        in_specs=[
            pl.BlockSpec((_M_BLOCK, SEMEME_SIZE), lambda i: (i, 0)),
            pl.BlockSpec((SEMEME_SIZE, D_MODEL), lambda i: (0, 0)),
            pl.BlockSpec(xt.shape, lambda i: (0, 0)),
        ],
        out_specs=[
            pl.BlockSpec((_M_BLOCK, D_MODEL), lambda i: (i, 0)),
            pl.BlockSpec((n_idx,), lambda i: (0,)),
        ],
        out_shape=[
            jax.ShapeDtypeStruct((VOCAB, D_MODEL), jnp.float32),
            jax.ShapeDtypeStruct((n_idx,), jnp.int32),
        ],
    )(word2sememe, lut_weight, xt)


def _sc_gather(table, idx):
    n = idx.shape[0]
    vocab = table.shape[0]
    b_per_w = n // _NW
    loaders = 5
    rows_per_loader = vocab // loaders
    mesh = plsc.VectorSubcoreMesh(core_axis_name="c", subcore_axis_name="s")

    n_chunks = 4
    chunk = b_per_w // n_chunks

    @pl.kernel(
        mesh=mesh,
        out_type=jax.ShapeDtypeStruct((n, D_MODEL), jnp.float32),
        scratch_types=[
            pltpu.VMEM((b_per_w,), jnp.int32),
            pltpu.VMEM((chunk, D_MODEL), jnp.float32),
            pltpu.VMEM((chunk, D_MODEL), jnp.float32),
            pltpu.VMEM_SHARED((vocab, D_MODEL), jnp.float32),
            pltpu.SemaphoreType.DMA,
            pltpu.SemaphoreType.DMA,
            pltpu.SemaphoreType.DMA,
            pltpu.SemaphoreType.DMA,
        ],
    )
    def k(table_hbm, idx_hbm, out_hbm, idx_v, buf0, buf1, table_sp, g0, g1, s0, s1):
        sid = lax.axis_index("s")
        wid = sid * _NC + lax.axis_index("c")
        base = wid * b_per_w
        idx_load = pltpu.async_copy(idx_hbm.at[pl.ds(base, b_per_w)], idx_v, g0)

        @pl.when(sid < loaders)
        def _():
            row0 = sid * rows_per_loader
            pltpu.sync_copy(
                table_hbm.at[pl.ds(row0, rows_per_loader)],
                table_sp.at[pl.ds(row0, rows_per_loader)],
            )

        plsc.subcore_barrier()
        idx_load.wait()

        bufs, gsems, ssems = (buf0, buf1), (g0, g1), (s0, s1)
        gathers = [None, None]
        stores = [None, None]
        gathers[0] = pltpu.async_copy(
            table_sp.at[idx_v.at[pl.ds(0, chunk)]], bufs[0], gsems[0]
        )
        for c in range(n_chunks):
            cur = c % 2
            nxt = (c + 1) % 2
            gathers[cur].wait()
            if c + 1 < n_chunks:
                if stores[nxt] is not None:
                    stores[nxt].wait()
                gathers[nxt] = pltpu.async_copy(
                    table_sp.at[idx_v.at[pl.ds((c + 1) * chunk, chunk)]],
                    bufs[nxt],
                    gsems[nxt],
                )
            stores[cur] = pltpu.async_copy(
                bufs[cur], out_hbm.at[pl.ds(base + c * chunk, chunk)], ssems[cur]
            )
        stores[0].wait()
        stores[1].wait()

    return k(table, idx)


def kernel(x, word2sememe, lut_weight):
    Bx, Lx = x.shape
    table, idx = _build_table(word2sememe, lut_weight, x.T)
    flat = _sc_gather(table, idx)
    return flat.reshape(Lx, Bx, D_MODEL).transpose(1, 0, 2)


# speedup vs baseline: 1.0253x; 1.0253x over previous
import math

import jax
import jax.numpy as jnp
from jax import lax
from jax.experimental import pallas as pl
from jax.experimental.pallas import tpu as pltpu
from jax.experimental.pallas import tpu_sc as plsc

VOCAB = 1000
SEMEME_SIZE = 2048
D_MODEL = 128

_NC = 2
_NS = 16
_NW = _NC * _NS


def _table_body(w2s_ref, lut_ref, xt_ref, t_ref, idx_ref):
    w = w2s_ref[...]
    s = jnp.sum(w, axis=1, keepdims=True) + 1e-6
    acc = jnp.dot(w, lut_ref[...], preferred_element_type=jnp.float32)
    t_ref[...] = acc * (math.sqrt(D_MODEL) / s)
    idx_ref[...] = xt_ref[...].reshape(-1)


def _build_table(word2sememe, lut_weight, xt):
    return pl.pallas_call(
        _table_body,
        out_shape=[
            jax.ShapeDtypeStruct((VOCAB, D_MODEL), jnp.float32),
            jax.ShapeDtypeStruct((xt.shape[0] * xt.shape[1],), jnp.int32),
        ],
    )(word2sememe, lut_weight, xt)


def _sc_gather(table, idx):
    n = idx.shape[0]
    vocab = table.shape[0]
    b_per_w = n // _NW
    loaders = 5
    rows_per_loader = vocab // loaders
    mesh = plsc.VectorSubcoreMesh(core_axis_name="c", subcore_axis_name="s")

    n_chunks = 2
    chunk = b_per_w // n_chunks

    @pl.kernel(
        mesh=mesh,
        out_type=jax.ShapeDtypeStruct((n, D_MODEL), jnp.float32),
        scratch_types=[
            pltpu.VMEM((b_per_w,), jnp.int32),
            pltpu.VMEM((chunk, D_MODEL), jnp.float32),
            pltpu.VMEM((chunk, D_MODEL), jnp.float32),
            pltpu.VMEM_SHARED((vocab, D_MODEL), jnp.float32),
            pltpu.SemaphoreType.DMA,
            pltpu.SemaphoreType.DMA,
            pltpu.SemaphoreType.DMA,
            pltpu.SemaphoreType.DMA,
        ],
    )
    def k(table_hbm, idx_hbm, out_hbm, idx_v, buf0, buf1, table_sp, g0, g1, s0, s1):
        sid = lax.axis_index("s")
        wid = sid * _NC + lax.axis_index("c")
        base = wid * b_per_w
        idx_load = pltpu.async_copy(idx_hbm.at[pl.ds(base, b_per_w)], idx_v, g0)

        @pl.when(sid < loaders)
        def _():
            row0 = sid * rows_per_loader
            pltpu.sync_copy(
                table_hbm.at[pl.ds(row0, rows_per_loader)],
                table_sp.at[pl.ds(row0, rows_per_loader)],
            )

        plsc.subcore_barrier()
        idx_load.wait()

        bufs, gsems, ssems = (buf0, buf1), (g0, g1), (s0, s1)
        gathers = [None, None]
        stores = [None, None]
        gathers[0] = pltpu.async_copy(
            table_sp.at[idx_v.at[pl.ds(0, chunk)]], bufs[0], gsems[0]
        )
        for c in range(n_chunks):
            cur = c % 2
            nxt = (c + 1) % 2
            gathers[cur].wait()
            if c + 1 < n_chunks:
                if stores[nxt] is not None:
                    stores[nxt].wait()
                gathers[nxt] = pltpu.async_copy(
                    table_sp.at[idx_v.at[pl.ds((c + 1) * chunk, chunk)]],
                    bufs[nxt],
                    gsems[nxt],
                )
            stores[cur] = pltpu.async_copy(
                bufs[cur], out_hbm.at[pl.ds(base + c * chunk, chunk)], ssems[cur]
            )
        stores[0].wait()
        stores[1].wait()

    return k(table, idx)


def kernel(x, word2sememe, lut_weight):
    Bx, Lx = x.shape
    table, idx = _build_table(word2sememe, lut_weight, x.T)
    flat = _sc_gather(table, idx)
    return flat.reshape(Lx, Bx, D_MODEL).transpose(1, 0, 2)


# speedup vs baseline: 1.0343x; 1.0088x over previous
import math

import jax
import jax.numpy as jnp
from jax import lax
from jax.experimental import pallas as pl
from jax.experimental.pallas import tpu as pltpu
from jax.experimental.pallas import tpu_sc as plsc

VOCAB = 1000
SEMEME_SIZE = 2048
D_MODEL = 128

_NC = 2
_NS = 16
_NW = _NC * _NS


def _table_body(w2s_ref, lut_ref, xt_ref, t_ref, idx_ref):
    w = w2s_ref[...]
    s = jnp.sum(w, axis=1, keepdims=True) + 1e-6
    acc = jnp.dot(w, lut_ref[...], preferred_element_type=jnp.float32)
    t_ref[...] = acc * (math.sqrt(D_MODEL) / s)
    idx_ref[...] = xt_ref[...].reshape(-1)


def _build_table(word2sememe, lut_weight, xt):
    return pl.pallas_call(
        _table_body,
        out_shape=[
            jax.ShapeDtypeStruct((VOCAB, D_MODEL), jnp.float32),
            jax.ShapeDtypeStruct((xt.shape[0] * xt.shape[1],), jnp.int32),
        ],
    )(word2sememe, lut_weight, xt)


def _sc_gather(table, idx):
    n = idx.shape[0]
    vocab = table.shape[0]
    b_per_w = n // _NW
    loaders = 5
    rows_per_loader = vocab // loaders
    mesh = plsc.VectorSubcoreMesh(core_axis_name="c", subcore_axis_name="s")

    n_chunks = 4
    chunk = b_per_w // n_chunks

    @pl.kernel(
        mesh=mesh,
        out_type=jax.ShapeDtypeStruct((n, D_MODEL), jnp.float32),
        scratch_types=[
            pltpu.VMEM((b_per_w,), jnp.int32),
            pltpu.VMEM((chunk, D_MODEL), jnp.float32),
            pltpu.VMEM((chunk, D_MODEL), jnp.float32),
            pltpu.VMEM_SHARED((vocab, D_MODEL), jnp.float32),
            pltpu.SemaphoreType.DMA,
            pltpu.SemaphoreType.DMA,
            pltpu.SemaphoreType.DMA,
            pltpu.SemaphoreType.DMA,
        ],
    )
    def k(table_hbm, idx_hbm, out_hbm, idx_v, buf0, buf1, table_sp, g0, g1, s0, s1):
        sid = lax.axis_index("s")
        wid = sid * _NC + lax.axis_index("c")
        base = wid * b_per_w
        idx_load = pltpu.async_copy(idx_hbm.at[pl.ds(base, b_per_w)], idx_v, g0)

        @pl.when(sid < loaders)
        def _():
            row0 = sid * rows_per_loader
            pltpu.sync_copy(
                table_hbm.at[pl.ds(row0, rows_per_loader)],
                table_sp.at[pl.ds(row0, rows_per_loader)],
            )

        plsc.subcore_barrier()
        idx_load.wait()

        bufs, gsems, ssems = (buf0, buf1), (g0, g1), (s0, s1)
        gathers = [None, None]
        stores = [None, None]
        gathers[0] = pltpu.async_copy(
            table_sp.at[idx_v.at[pl.ds(0, chunk)]], bufs[0], gsems[0]
        )
        for c in range(n_chunks):
            cur = c % 2
            nxt = (c + 1) % 2
            gathers[cur].wait()
            if c + 1 < n_chunks:
                if stores[nxt] is not None:
                    stores[nxt].wait()
                gathers[nxt] = pltpu.async_copy(
                    table_sp.at[idx_v.at[pl.ds((c + 1) * chunk, chunk)]],
                    bufs[nxt],
                    gsems[nxt],
                )
            stores[cur] = pltpu.async_copy(
                bufs[cur], out_hbm.at[pl.ds(base + c * chunk, chunk)], ssems[cur]
            )
        stores[0].wait()
        stores[1].wait()

    return k(table, idx)


def kernel(x, word2sememe, lut_weight):
    Bx, Lx = x.shape
    table, idx = _build_table(word2sememe, lut_weight, x.T)
    flat = _sc_gather(table, idx)
    return flat.reshape(Lx, Bx, D_MODEL).transpose(1, 0, 2)
